# baseline (speedup 1.0000x reference)
import jax
import jax.numpy as jnp
from jax import lax
from jax.experimental import pallas as pl
from jax.experimental.pallas import tpu as pltpu
from jax.experimental.pallas import tpu_sc as plsc

B = 8
V = 16
NIN = 4
NTGT = 8
POSE_W = 16
HH = 128
CPC = 256 // HH
NCH = 3 * CPC
NBUF = 3

_mesh = plsc.VectorSubcoreMesh(core_axis_name="c", subcore_axis_name="s")


def _sc_split(image, idxflat):
    def body(img_ref, idx_hbm, tgt_img, idx_v, *ring):
        bufs = ring[:NBUF]
        sem = ring[NBUF]
        isems = ring[NBUF + 1:2 * NBUF + 1]
        osems = ring[2 * NBUF + 1:]

        cid = lax.axis_index("c")
        sid = lax.axis_index("s")
        w = sid * 2 + cid

        pltpu.sync_copy(idx_hbm, idx_v.at[pl.ds(0, B * NTGT)])

        e0 = 2 * w
        vec = idx_v[pl.ds(e0, 16)]
        v0 = vec[0]
        v1 = vec[1]
        b0 = e0 // NTGT
        b1 = (e0 + 1) // NTGT
        t0 = e0 % NTGT
        t1 = (e0 + 1) % NTGT

        tasks = []
        for src_b, src_v, dst_ref, dst_b, dst_v in (
                (b0, v0, tgt_img, b0, t0),
                (b1, v1, tgt_img, b1, t1)):
            for c in range(NCH):
                ci, h0 = c // CPC, (c % CPC) * HH
                tasks.append((src_b, src_v, dst_ref, dst_b, dst_v, ci, h0))
        n = len(tasks)

        in_h = [None] * NBUF
        out_h = [None] * NBUF
        for i in range(-(NBUF - 1), n):
            j = i + NBUF - 1
            if j < n:
                k2 = j % NBUF
                if j >= NBUF:
                    out_h[k2].wait()
                sb, sv, _, _, _, ci, h0 = tasks[j]
                in_h[k2] = pltpu.async_copy(
                    img_ref.at[sb, sv, ci, pl.ds(h0, HH)], bufs[k2], isems[k2])
            if i >= 0:
                k = i % NBUF
                in_h[k].wait()
                _, _, dst_ref, db, dv, ci, h0 = tasks[i]
                out_h[k] = pltpu.async_copy(
                    bufs[k], dst_ref.at[db, dv, ci, pl.ds(h0, HH)], osems[k])
        for k in range(NBUF):
            out_h[k].wait()

    f = pl.kernel(
        body,
        out_type=[
            jax.ShapeDtypeStruct((B, NTGT, 3, 256, 256), jnp.float32),
        ],
        mesh=_mesh,
        compiler_params=pltpu.CompilerParams(use_tc_tiling_on_sc=True),
        scratch_types=[
            pltpu.VMEM((B * NTGT + 16,), jnp.int32),
            *[pltpu.VMEM((HH, 256), jnp.float32) for _ in range(NBUF)],
            *[pltpu.SemaphoreType.DMA for _ in range(2 * NBUF + 1)],
        ],
    )
    return f(image, idxflat)


def _tc_slice(image, pose, idxflat):
    def body(idx_s, img_blk, pose_blk, out_blk, in_pose_blk, tgt_pose_blk):
        out_blk[...] = img_blk[...]

        @pl.when(pl.program_id(0) == 0)
        def _():
            in_pose_blk[...] = pose_blk[:, :NIN]
            for b in range(B):
                for t in range(NTGT):
                    v = idx_s[b * NTGT + t]
                    tgt_pose_blk[b, t] = pose_blk[b, v]

    return pl.pallas_call(
        body,
        grid=(B * NIN,),
        in_specs=[
            pl.BlockSpec(memory_space=pltpu.SMEM),
            pl.BlockSpec(
                (1, 1, 3, 256, 256), lambda i: (i // NIN, i % NIN, 0, 0, 0)),
            pl.BlockSpec((B, V, 4, 4), lambda i: (0, 0, 0, 0)),
        ],
        out_specs=[
            pl.BlockSpec(
                (1, 1, 3, 256, 256), lambda i: (i // NIN, i % NIN, 0, 0, 0)),
            pl.BlockSpec((B, NIN, 4, 4), lambda i: (0, 0, 0, 0)),
            pl.BlockSpec((B, NTGT, 4, 4), lambda i: (0, 0, 0, 0)),
        ],
        out_shape=[
            jax.ShapeDtypeStruct((B, NIN, 3, 256, 256), jnp.float32),
            jax.ShapeDtypeStruct((B, NIN, 4, 4), jnp.float32),
            jax.ShapeDtypeStruct((B, NTGT, 4, 4), jnp.float32),
        ],
    )(idxflat, image, pose)


def _tc_gather(image, idxflat):
    def body(idx_s, img_blk, out_blk):
        out_blk[...] = img_blk[...]

    grid_spec = pltpu.PrefetchScalarGridSpec(
        num_scalar_prefetch=1,
        grid=(B * NTGT,),
        in_specs=[pl.BlockSpec(
            (1, 1, 3, 256, 256), lambda i, idx: (i // NTGT, idx[i], 0, 0, 0))],
        out_specs=pl.BlockSpec(
            (1, 1, 3, 256, 256), lambda i, idx: (i // NTGT, i % NTGT, 0, 0, 0)),
    )
    return pl.pallas_call(
        body,
        grid_spec=grid_spec,
        out_shape=jax.ShapeDtypeStruct((B, NTGT, 3, 256, 256), jnp.float32),
    )(idxflat, image)


def kernel(image, pose, index):
    idxflat = index.reshape(B * NTGT).astype(jnp.int32)
    in_img, in_pose, tgt_pose = _tc_slice(image, pose, idxflat)
    tgt_img = _tc_gather(image, idxflat)
    return (in_img, in_pose, tgt_img, tgt_pose)


# speedup vs baseline: 1.2886x; 1.2886x over previous
import jax
import jax.numpy as jnp
from jax import lax
from jax.experimental import pallas as pl
from jax.experimental.pallas import tpu as pltpu
from jax.experimental.pallas import tpu_sc as plsc

B = 8
V = 16
NIN = 4
NTGT = 8
POSE_W = 16
HH = 64
CPC = 256 // HH
NCH = 3 * CPC
NBUF = 4

_mesh = plsc.VectorSubcoreMesh(core_axis_name="c", subcore_axis_name="s")


def _sc_split(image, idxflat):
    def body(img_ref, idx_hbm, tgt_img, idx_v, *ring):
        bufs = ring[:NBUF]
        sem = ring[NBUF]
        isems = ring[NBUF + 1:2 * NBUF + 1]
        osems = ring[2 * NBUF + 1:]

        cid = lax.axis_index("c")
        sid = lax.axis_index("s")
        w = sid * 2 + cid

        pltpu.sync_copy(idx_hbm, idx_v.at[pl.ds(0, B * NTGT)])

        e0 = 2 * w
        vec = idx_v[pl.ds(e0, 16)]
        v0 = vec[0]
        v1 = vec[1]
        b0 = e0 // NTGT
        b1 = (e0 + 1) // NTGT
        t0 = e0 % NTGT
        t1 = (e0 + 1) % NTGT

        tasks = []
        for src_b, src_v, dst_ref, dst_b, dst_v in (
                (b0, v0, tgt_img, b0, t0),
                (b1, v1, tgt_img, b1, t1)):
            for c in range(NCH):
                ci, h0 = c // CPC, (c % CPC) * HH
                tasks.append((src_b, src_v, dst_ref, dst_b, dst_v, ci, h0))
        n = len(tasks)

        in_h = [None] * NBUF
        out_h = [None] * NBUF
        for i in range(-(NBUF - 1), n):
            j = i + NBUF - 1
            if j < n:
                k2 = j % NBUF
                if j >= NBUF:
                    out_h[k2].wait()
                sb, sv, _, _, _, ci, h0 = tasks[j]
                in_h[k2] = pltpu.async_copy(
                    img_ref.at[sb, sv, ci, pl.ds(h0, HH)], bufs[k2], isems[k2])
            if i >= 0:
                k = i % NBUF
                in_h[k].wait()
                _, _, dst_ref, db, dv, ci, h0 = tasks[i]
                out_h[k] = pltpu.async_copy(
                    bufs[k], dst_ref.at[db, dv, ci, pl.ds(h0, HH)], osems[k])
        for k in range(NBUF):
            out_h[k].wait()

    f = pl.kernel(
        body,
        out_type=[
            jax.ShapeDtypeStruct((B, NTGT, 3, 256, 256), jnp.float32),
        ],
        mesh=_mesh,
        compiler_params=pltpu.CompilerParams(use_tc_tiling_on_sc=True),
        scratch_types=[
            pltpu.VMEM((B * NTGT + 16,), jnp.int32),
            *[pltpu.VMEM((HH, 256), jnp.float32) for _ in range(NBUF)],
            *[pltpu.SemaphoreType.DMA for _ in range(2 * NBUF + 1)],
        ],
    )
    return f(image, idxflat)


def _tc_slice(image, pose, idxflat):
    def body(idx_s, img_blk, pose_blk, out_blk, in_pose_blk, tgt_pose_blk):
        out_blk[...] = img_blk[...]

        @pl.when(pl.program_id(0) == 0)
        def _():
            in_pose_blk[...] = pose_blk[:, :NIN]
            for b in range(B):
                for t in range(NTGT):
                    v = idx_s[b * NTGT + t]
                    tgt_pose_blk[b, t] = pose_blk[b, v]

    return pl.pallas_call(
        body,
        grid=(B * NIN,),
        in_specs=[
            pl.BlockSpec(memory_space=pltpu.SMEM),
            pl.BlockSpec(
                (1, 1, 3, 256, 256), lambda i: (i // NIN, i % NIN, 0, 0, 0)),
            pl.BlockSpec((B, V, 4, 4), lambda i: (0, 0, 0, 0)),
        ],
        out_specs=[
            pl.BlockSpec(
                (1, 1, 3, 256, 256), lambda i: (i // NIN, i % NIN, 0, 0, 0)),
            pl.BlockSpec((B, NIN, 4, 4), lambda i: (0, 0, 0, 0)),
            pl.BlockSpec((B, NTGT, 4, 4), lambda i: (0, 0, 0, 0)),
        ],
        out_shape=[
            jax.ShapeDtypeStruct((B, NIN, 3, 256, 256), jnp.float32),
            jax.ShapeDtypeStruct((B, NIN, 4, 4), jnp.float32),
            jax.ShapeDtypeStruct((B, NTGT, 4, 4), jnp.float32),
        ],
    )(idxflat, image, pose)


def kernel(image, pose, index):
    idxflat = index.reshape(B * NTGT).astype(jnp.int32)
    in_img, in_pose, tgt_pose = _tc_slice(image, pose, idxflat)
    (tgt_img,) = _sc_split(image, idxflat)
    return (in_img, in_pose, tgt_img, tgt_pose)


# speedup vs baseline: 1.2919x; 1.0025x over previous
import jax
import jax.numpy as jnp
from jax import lax
from jax.experimental import pallas as pl
from jax.experimental.pallas import tpu as pltpu
from jax.experimental.pallas import tpu_sc as plsc

B = 8
V = 16
NIN = 4
NTGT = 8
POSE_W = 16
HH = 128
CPC = 256 // HH
NCH = 3 * CPC
NBUF = 3

_mesh = plsc.VectorSubcoreMesh(core_axis_name="c", subcore_axis_name="s")


def _sc_split(image, idxflat):
    def body(img_ref, idx_hbm, tgt_img, idx_v, *ring):
        bufs = ring[:NBUF]
        sem = ring[NBUF]
        isems = ring[NBUF + 1:2 * NBUF + 1]
        osems = ring[2 * NBUF + 1:]

        cid = lax.axis_index("c")
        sid = lax.axis_index("s")
        w = sid * 2 + cid

        pltpu.sync_copy(idx_hbm, idx_v.at[pl.ds(0, B * NTGT)])

        e0 = 2 * w
        vec = idx_v[pl.ds(e0, 16)]
        v0 = vec[0]
        v1 = vec[1]
        b0 = e0 // NTGT
        b1 = (e0 + 1) // NTGT
        t0 = e0 % NTGT
        t1 = (e0 + 1) % NTGT

        tasks = []
        for src_b, src_v, dst_ref, dst_b, dst_v in (
                (b0, v0, tgt_img, b0, t0),
                (b1, v1, tgt_img, b1, t1)):
            for c in range(NCH):
                ci, h0 = c // CPC, (c % CPC) * HH
                tasks.append((src_b, src_v, dst_ref, dst_b, dst_v, ci, h0))
        n = len(tasks)

        in_h = [None] * NBUF
        out_h = [None] * NBUF
        for i in range(-(NBUF - 1), n):
            j = i + NBUF - 1
            if j < n:
                k2 = j % NBUF
                if j >= NBUF:
                    out_h[k2].wait()
                sb, sv, _, _, _, ci, h0 = tasks[j]
                in_h[k2] = pltpu.async_copy(
                    img_ref.at[sb, sv, ci, pl.ds(h0, HH)], bufs[k2], isems[k2])
            if i >= 0:
                k = i % NBUF
                in_h[k].wait()
                _, _, dst_ref, db, dv, ci, h0 = tasks[i]
                out_h[k] = pltpu.async_copy(
                    bufs[k], dst_ref.at[db, dv, ci, pl.ds(h0, HH)], osems[k])
        for k in range(NBUF):
            out_h[k].wait()

    f = pl.kernel(
        body,
        out_type=[
            jax.ShapeDtypeStruct((B, NTGT, 3, 256, 256), jnp.float32),
        ],
        mesh=_mesh,
        compiler_params=pltpu.CompilerParams(use_tc_tiling_on_sc=True),
        scratch_types=[
            pltpu.VMEM((B * NTGT + 16,), jnp.int32),
            *[pltpu.VMEM((HH, 256), jnp.float32) for _ in range(NBUF)],
            *[pltpu.SemaphoreType.DMA for _ in range(2 * NBUF + 1)],
        ],
    )
    return f(image, idxflat)


def _tc_slice(image, pose, idxflat):
    def body(idx_s, img_blk, pose_blk, out_blk, in_pose_blk, tgt_pose_blk):
        out_blk[...] = img_blk[...]

        @pl.when(pl.program_id(0) == 0)
        def _():
            in_pose_blk[...] = pose_blk[:, :NIN]
            for b in range(B):
                for t in range(NTGT):
                    v = idx_s[b * NTGT + t]
                    tgt_pose_blk[b, t] = pose_blk[b, v]

    return pl.pallas_call(
        body,
        grid=(B * NIN,),
        in_specs=[
            pl.BlockSpec(memory_space=pltpu.SMEM),
            pl.BlockSpec(
                (1, 1, 3, 256, 256), lambda i: (i // NIN, i % NIN, 0, 0, 0)),
            pl.BlockSpec((B, V, 4, 4), lambda i: (0, 0, 0, 0)),
        ],
        out_specs=[
            pl.BlockSpec(
                (1, 1, 3, 256, 256), lambda i: (i // NIN, i % NIN, 0, 0, 0)),
            pl.BlockSpec((B, NIN, 4, 4), lambda i: (0, 0, 0, 0)),
            pl.BlockSpec((B, NTGT, 4, 4), lambda i: (0, 0, 0, 0)),
        ],
        out_shape=[
            jax.ShapeDtypeStruct((B, NIN, 3, 256, 256), jnp.float32),
            jax.ShapeDtypeStruct((B, NIN, 4, 4), jnp.float32),
            jax.ShapeDtypeStruct((B, NTGT, 4, 4), jnp.float32),
        ],
    )(idxflat, image, pose)


def kernel(image, pose, index):
    idxflat = index.reshape(B * NTGT).astype(jnp.int32)
    in_img, in_pose, tgt_pose = _tc_slice(image, pose, idxflat)
    (tgt_img,) = _sc_split(image, idxflat)
    return (in_img, in_pose, tgt_img, tgt_pose)
